# baseline (speedup 1.0000x reference)
import functools

import jax
import jax.numpy as jnp
from jax import lax
from jax.experimental import pallas as pl
from jax.experimental.pallas import tpu as pltpu
from jax.experimental.pallas import tpu_sc as plsc

BATCH = 16384
DIM = 64
NUM_WORKERS = 32
B_PER_W = BATCH // NUM_WORKERS
GROUPS = B_PER_W // 16


def _body(user_table, video_table, user_idx, video_idx, out_hbm,
          idx_u, idx_v, rows_u, rows_v, out_v, sem_u, sem_v):
    wid = lax.axis_index("s") * 2 + lax.axis_index("c")
    base = wid * B_PER_W

    pltpu.sync_copy(user_idx.at[pl.ds(base, B_PER_W)], idx_u)
    pltpu.sync_copy(video_idx.at[pl.ds(base, B_PER_W)], idx_v)

    cp_u = pltpu.async_copy(user_table.at[idx_u], rows_u, sem_u)
    cp_v = pltpu.async_copy(video_table.at[idx_v], rows_v, sem_v)
    cp_u.wait()
    cp_v.wait()

    lane = lax.iota(jnp.int32, 16)

    def group(g, carry):
        row_idx = g * 16 + lane
        acc = jnp.zeros((16,), jnp.float32)
        for j in range(DIM):
            col_idx = jnp.full((16,), j, jnp.int32)
            u = plsc.load_gather(rows_u, [row_idx, col_idx])
            v = plsc.load_gather(rows_v, [row_idx, col_idx])
            acc = acc + u * v
        out_v[pl.ds(g * 16, 16)] = acc
        return carry

    lax.fori_loop(0, GROUPS, group, 0)

    pltpu.sync_copy(out_v, out_hbm.at[pl.ds(base, B_PER_W)])


@jax.jit
def kernel(user_idx, video_idx, user_table, video_table):
    mesh = plsc.VectorSubcoreMesh(core_axis_name="c", subcore_axis_name="s")
    k = functools.partial(
        pl.kernel,
        mesh=mesh,
        out_type=jax.ShapeDtypeStruct((BATCH,), jnp.float32),
        scratch_types=[
            pltpu.VMEM((B_PER_W,), jnp.int32),
            pltpu.VMEM((B_PER_W,), jnp.int32),
            pltpu.VMEM((B_PER_W, DIM), jnp.float32),
            pltpu.VMEM((B_PER_W, DIM), jnp.float32),
            pltpu.VMEM((B_PER_W,), jnp.float32),
            pltpu.SemaphoreType.DMA,
            pltpu.SemaphoreType.DMA,
        ],
        compiler_params=pltpu.CompilerParams(
            needs_layout_passes=False, use_tc_tiling_on_sc=False),
    )(_body)
    return k(user_table, video_table,
             user_idx.astype(jnp.int32), video_idx.astype(jnp.int32))
